# baseline (speedup 1.0000x reference)
import jax
import jax.numpy as jnp
from jax.experimental import pallas as pl

N = 10000
E = 160000
D = 256
NE = 8
RB = 1000

_INTERP = False


def _gate_body(x_ref, wg_ref, bg_ref, p_ref, oh0_ref, oh1_ref, ep_ref):
    logits = jnp.dot(x_ref[...], wg_ref[...],
                     preferred_element_type=jnp.float32) + bg_ref[...]
    m = jnp.max(logits, axis=1, keepdims=True)
    ex = jnp.exp(logits - m)
    p = ex / jnp.sum(ex, axis=1, keepdims=True)
    iota = jax.lax.broadcasted_iota(jnp.int32, p.shape, 1)
    m0 = jnp.max(p, axis=1, keepdims=True)
    i0 = jnp.min(jnp.where(p == m0, iota, NE + 1), axis=1, keepdims=True)
    oh0 = (iota == i0).astype(jnp.float32)
    p1 = jnp.where(iota == i0, -1.0, p)
    m1 = jnp.max(p1, axis=1, keepdims=True)
    i1 = jnp.min(jnp.where(p1 == m1, iota, NE + 1), axis=1, keepdims=True)
    oh1 = (iota == i1).astype(jnp.float32)
    p_ref[...] = p
    oh0_ref[...] = oh0
    oh1_ref[...] = oh1
    ep_ref[...] = jnp.where(iota == 0, i0, jnp.where(iota == 1, i1, 0))


def _gate(x, Wg, bg):
    nb = N // RB
    return pl.pallas_call(
        _gate_body,
        grid=(nb,),
        in_specs=[
            pl.BlockSpec((RB, D), lambda r: (r, 0)),
            pl.BlockSpec((D, NE), lambda r: (0, 0)),
            pl.BlockSpec((NE,), lambda r: (0,)),
        ],
        out_specs=[
            pl.BlockSpec((RB, NE), lambda r: (r, 0)),
            pl.BlockSpec((RB, NE), lambda r: (r, 0)),
            pl.BlockSpec((RB, NE), lambda r: (r, 0)),
            pl.BlockSpec((RB, NE), lambda r: (r, 0)),
        ],
        out_shape=[
            jax.ShapeDtypeStruct((N, NE), jnp.float32),
            jax.ShapeDtypeStruct((N, NE), jnp.float32),
            jax.ShapeDtypeStruct((N, NE), jnp.float32),
            jax.ShapeDtypeStruct((N, NE), jnp.int32),
        ],
        interpret=_INTERP,
    )(x, Wg, bg)


def _expert_body(xcat_ref, w1_ref, b1_ref, wl2_ref, wr2_ref, oh0_ref, oh1_ref,
                 h1wlo_ref, h1whi_ref, h1r0_ref, h1r1_ref):
    e = pl.program_id(1)
    h1 = jnp.dot(xcat_ref[...], w1_ref[0],
                 preferred_element_type=jnp.float32) + b1_ref[0]
    h1 = jnp.maximum(h1, 0.0)
    h1w = jnp.dot(h1, wl2_ref[0], preferred_element_type=jnp.float32)
    h1wlo_ref[0] = h1w[:, :128]
    h1whi_ref[0] = h1w[:, 128:]
    hr = jnp.dot(h1, wr2_ref[0], preferred_element_type=jnp.float32)
    iota = jax.lax.broadcasted_iota(jnp.int32, oh0_ref.shape, 1)
    sel = (iota == e).astype(jnp.float32)
    m0 = jnp.sum(oh0_ref[...] * sel, axis=1, keepdims=True)
    m1 = jnp.sum(oh1_ref[...] * sel, axis=1, keepdims=True)

    @pl.when(e == 0)
    def _():
        h1r0_ref[...] = m0 * hr
        h1r1_ref[...] = m1 * hr

    @pl.when(e > 0)
    def _():
        h1r0_ref[...] += m0 * hr
        h1r1_ref[...] += m1 * hr


def _expert_mats(xcat, W1cat, bl1, Wl2, Wr2, oh0, oh1):
    nb = N // RB
    return pl.pallas_call(
        _expert_body,
        grid=(nb, NE),
        in_specs=[
            pl.BlockSpec((RB, 2 * D), lambda r, e: (r, 0)),
            pl.BlockSpec((1, 2 * D, D), lambda r, e: (e, 0, 0)),
            pl.BlockSpec((1, 1, D), lambda r, e: (e, 0, 0)),
            pl.BlockSpec((1, D, D), lambda r, e: (e, 0, 0)),
            pl.BlockSpec((1, D, D), lambda r, e: (e, 0, 0)),
            pl.BlockSpec((RB, NE), lambda r, e: (r, 0)),
            pl.BlockSpec((RB, NE), lambda r, e: (r, 0)),
        ],
        out_specs=[
            pl.BlockSpec((1, RB, 128), lambda r, e: (e, r, 0)),
            pl.BlockSpec((1, RB, 128), lambda r, e: (e, r, 0)),
            pl.BlockSpec((RB, D), lambda r, e: (r, 0)),
            pl.BlockSpec((RB, D), lambda r, e: (r, 0)),
        ],
        out_shape=[
            jax.ShapeDtypeStruct((NE, N, 128), jnp.float32),
            jax.ShapeDtypeStruct((NE, N, 128), jnp.float32),
            jax.ShapeDtypeStruct((N, D), jnp.float32),
            jax.ShapeDtypeStruct((N, D), jnp.float32),
        ],
        interpret=_INTERP,
    )(xcat, W1cat, bl1, Wl2, Wr2, oh0, oh1)


def _combine_body(a0_ref, a1_ref, rdeg_ref, p_ref, oh0_ref, oh1_ref, bl2_ref,
                  h1r0_ref, h1r1_ref, out_ref):
    rdeg = rdeg_ref[...]
    w0 = jnp.sum(p_ref[...] * oh0_ref[...], axis=1, keepdims=True)
    w1 = jnp.sum(p_ref[...] * oh1_ref[...], axis=1, keepdims=True)
    b0 = jnp.dot(oh0_ref[...], bl2_ref[...], preferred_element_type=jnp.float32)
    b1 = jnp.dot(oh1_ref[...], bl2_ref[...], preferred_element_type=jnp.float32)
    o0 = jnp.maximum(a0_ref[...] * rdeg + b0 + h1r0_ref[...], 0.0)
    o1 = jnp.maximum(a1_ref[...] * rdeg + b1 + h1r1_ref[...], 0.0)
    out_ref[...] = w0 * o0 + w1 * o1


def _combine(a0, a1, rdeg, p, oh0, oh1, bl2, h1r0, h1r1):
    nb = N // RB
    return pl.pallas_call(
        _combine_body,
        grid=(nb,),
        in_specs=[
            pl.BlockSpec((RB, D), lambda r: (r, 0)),
            pl.BlockSpec((RB, D), lambda r: (r, 0)),
            pl.BlockSpec((RB, 1), lambda r: (r, 0)),
            pl.BlockSpec((RB, NE), lambda r: (r, 0)),
            pl.BlockSpec((RB, NE), lambda r: (r, 0)),
            pl.BlockSpec((RB, NE), lambda r: (r, 0)),
            pl.BlockSpec((NE, D), lambda r: (0, 0)),
            pl.BlockSpec((RB, D), lambda r: (r, 0)),
            pl.BlockSpec((RB, D), lambda r: (r, 0)),
        ],
        out_specs=pl.BlockSpec((RB, D), lambda r: (r, 0)),
        out_shape=jax.ShapeDtypeStruct((N, D), jnp.float32),
        interpret=_INTERP,
    )(a0, a1, rdeg, p, oh0, oh1, bl2, h1r0, h1r1)


def kernel(x, edge_index, Wg, bg, Wl1, bl1, Wr1, Wl2, bl2, Wr2):
    src = edge_index[0]
    dst = edge_index[1]

    p, oh0, oh1, ep = _gate(x, Wg, bg)
    e0 = ep[:, 0]
    e1 = ep[:, 1]

    ones = jnp.ones((E,), jnp.float32)
    deg = jax.ops.segment_sum(ones, dst, num_segments=N)
    xlo = x[:, :128]
    xhi = x[:, 128:]

    def seg(tbl, idx):
        return jax.ops.segment_sum(jnp.take(tbl, idx, axis=0), dst,
                                   num_segments=N)

    aggx = jnp.concatenate([seg(xlo, src), seg(xhi, src)], axis=1)
    rdeg = (1.0 / jnp.maximum(deg, 1.0))[:, None]
    meanx = aggx * rdeg

    xcat = jnp.concatenate([meanx, x], axis=1)
    W1cat = jnp.concatenate([Wl1, Wr1], axis=1)
    h1wlo, h1whi, h1r0, h1r1 = _expert_mats(xcat, W1cat, bl1[:, None, :],
                                            Wl2, Wr2, oh0, oh1)

    tlo = h1wlo.reshape(NE * N, 128)
    thi = h1whi.reshape(NE * N, 128)
    g0 = e0[dst] * N + src
    g1 = e1[dst] * N + src
    a0 = jnp.concatenate([seg(tlo, g0), seg(thi, g0)], axis=1)
    a1 = jnp.concatenate([seg(tlo, g1), seg(thi, g1)], axis=1)

    return _combine(a0, a1, rdeg, p, oh0, oh1, bl2, h1r0, h1r1)


# speedup vs baseline: 1.3091x; 1.3091x over previous
import jax
import jax.numpy as jnp
from jax.experimental import pallas as pl

N = 10000
E = 160000
D = 256
NE = 8
RB = 1000

_INTERP = False


def _gate_body(x_ref, wg_ref, bg_ref, p_ref, oh0_ref, oh1_ref, ep_ref):
    logits = jnp.dot(x_ref[...], wg_ref[...],
                     preferred_element_type=jnp.float32) + bg_ref[...]
    m = jnp.max(logits, axis=1, keepdims=True)
    ex = jnp.exp(logits - m)
    p = ex / jnp.sum(ex, axis=1, keepdims=True)
    iota = jax.lax.broadcasted_iota(jnp.int32, p.shape, 1)
    m0 = jnp.max(p, axis=1, keepdims=True)
    i0 = jnp.min(jnp.where(p == m0, iota, NE + 1), axis=1, keepdims=True)
    oh0 = (iota == i0).astype(jnp.float32)
    p1 = jnp.where(iota == i0, -1.0, p)
    m1 = jnp.max(p1, axis=1, keepdims=True)
    i1 = jnp.min(jnp.where(p1 == m1, iota, NE + 1), axis=1, keepdims=True)
    oh1 = (iota == i1).astype(jnp.float32)
    p_ref[...] = p
    oh0_ref[...] = oh0
    oh1_ref[...] = oh1
    ep_ref[...] = jnp.where(iota == 0, i0, jnp.where(iota == 1, i1, 0))


def _gate(x, Wg, bg):
    nb = N // RB
    return pl.pallas_call(
        _gate_body,
        grid=(nb,),
        in_specs=[
            pl.BlockSpec((RB, D), lambda r: (r, 0)),
            pl.BlockSpec((D, NE), lambda r: (0, 0)),
            pl.BlockSpec((NE,), lambda r: (0,)),
        ],
        out_specs=[
            pl.BlockSpec((RB, NE), lambda r: (r, 0)),
            pl.BlockSpec((RB, NE), lambda r: (r, 0)),
            pl.BlockSpec((RB, NE), lambda r: (r, 0)),
            pl.BlockSpec((RB, NE), lambda r: (r, 0)),
        ],
        out_shape=[
            jax.ShapeDtypeStruct((N, NE), jnp.float32),
            jax.ShapeDtypeStruct((N, NE), jnp.float32),
            jax.ShapeDtypeStruct((N, NE), jnp.float32),
            jax.ShapeDtypeStruct((N, NE), jnp.int32),
        ],
        interpret=_INTERP,
    )(x, Wg, bg)


def _expert_body(xcat_ref, w1_ref, b1_ref, wl2_ref, wr2_ref, oh0_ref, oh1_ref,
                 h1w_ref, h1r0_ref, h1r1_ref):
    e = pl.program_id(1)
    h1 = jnp.dot(xcat_ref[...], w1_ref[0],
                 preferred_element_type=jnp.float32) + b1_ref[0]
    h1 = jnp.maximum(h1, 0.0)
    h1w_ref[0] = jnp.dot(h1, wl2_ref[0], preferred_element_type=jnp.float32)
    hr = jnp.dot(h1, wr2_ref[0], preferred_element_type=jnp.float32)
    iota = jax.lax.broadcasted_iota(jnp.int32, oh0_ref.shape, 1)
    sel = (iota == e).astype(jnp.float32)
    m0 = jnp.sum(oh0_ref[...] * sel, axis=1, keepdims=True)
    m1 = jnp.sum(oh1_ref[...] * sel, axis=1, keepdims=True)

    @pl.when(e == 0)
    def _():
        h1r0_ref[...] = m0 * hr
        h1r1_ref[...] = m1 * hr

    @pl.when(e > 0)
    def _():
        h1r0_ref[...] += m0 * hr
        h1r1_ref[...] += m1 * hr


def _expert_mats(xcat, W1cat, bl1, Wl2, Wr2, oh0, oh1):
    nb = N // RB
    return pl.pallas_call(
        _expert_body,
        grid=(nb, NE),
        in_specs=[
            pl.BlockSpec((RB, 2 * D), lambda r, e: (r, 0)),
            pl.BlockSpec((1, 2 * D, D), lambda r, e: (e, 0, 0)),
            pl.BlockSpec((1, 1, D), lambda r, e: (e, 0, 0)),
            pl.BlockSpec((1, D, D), lambda r, e: (e, 0, 0)),
            pl.BlockSpec((1, D, D), lambda r, e: (e, 0, 0)),
            pl.BlockSpec((RB, NE), lambda r, e: (r, 0)),
            pl.BlockSpec((RB, NE), lambda r, e: (r, 0)),
        ],
        out_specs=[
            pl.BlockSpec((1, RB, D), lambda r, e: (e, r, 0)),
            pl.BlockSpec((RB, D), lambda r, e: (r, 0)),
            pl.BlockSpec((RB, D), lambda r, e: (r, 0)),
        ],
        out_shape=[
            jax.ShapeDtypeStruct((NE, N, D), jnp.float32),
            jax.ShapeDtypeStruct((N, D), jnp.float32),
            jax.ShapeDtypeStruct((N, D), jnp.float32),
        ],
        interpret=_INTERP,
    )(xcat, W1cat, bl1, Wl2, Wr2, oh0, oh1)


def _combine_body(a0_ref, a1_ref, rdeg_ref, p_ref, oh0_ref, oh1_ref, bl2_ref,
                  h1r0_ref, h1r1_ref, out_ref):
    rdeg = rdeg_ref[...]
    w0 = jnp.sum(p_ref[...] * oh0_ref[...], axis=1, keepdims=True)
    w1 = jnp.sum(p_ref[...] * oh1_ref[...], axis=1, keepdims=True)
    b0 = jnp.dot(oh0_ref[...], bl2_ref[...], preferred_element_type=jnp.float32)
    b1 = jnp.dot(oh1_ref[...], bl2_ref[...], preferred_element_type=jnp.float32)
    o0 = jnp.maximum(a0_ref[...] * rdeg + b0 + h1r0_ref[...], 0.0)
    o1 = jnp.maximum(a1_ref[...] * rdeg + b1 + h1r1_ref[...], 0.0)
    out_ref[...] = w0 * o0 + w1 * o1


def _combine(a0, a1, rdeg, p, oh0, oh1, bl2, h1r0, h1r1):
    nb = N // RB
    return pl.pallas_call(
        _combine_body,
        grid=(nb,),
        in_specs=[
            pl.BlockSpec((RB, D), lambda r: (r, 0)),
            pl.BlockSpec((RB, D), lambda r: (r, 0)),
            pl.BlockSpec((RB, 1), lambda r: (r, 0)),
            pl.BlockSpec((RB, NE), lambda r: (r, 0)),
            pl.BlockSpec((RB, NE), lambda r: (r, 0)),
            pl.BlockSpec((RB, NE), lambda r: (r, 0)),
            pl.BlockSpec((NE, D), lambda r: (0, 0)),
            pl.BlockSpec((RB, D), lambda r: (r, 0)),
            pl.BlockSpec((RB, D), lambda r: (r, 0)),
        ],
        out_specs=pl.BlockSpec((RB, D), lambda r: (r, 0)),
        out_shape=jax.ShapeDtypeStruct((N, D), jnp.float32),
        interpret=_INTERP,
    )(a0, a1, rdeg, p, oh0, oh1, bl2, h1r0, h1r1)


def kernel(x, edge_index, Wg, bg, Wl1, bl1, Wr1, Wl2, bl2, Wr2):
    src = edge_index[0]
    dst = edge_index[1]

    p, oh0, oh1, ep = _gate(x, Wg, bg)
    e0 = ep[:, 0]
    e1 = ep[:, 1]

    ones = jnp.ones((E,), jnp.float32)
    deg = jax.ops.segment_sum(ones, dst, num_segments=N)
    aggx = jax.ops.segment_sum(jnp.take(x, src, axis=0), dst, num_segments=N)
    rdeg = (1.0 / jnp.maximum(deg, 1.0))[:, None]
    meanx = aggx * rdeg

    xcat = jnp.concatenate([meanx, x], axis=1)
    W1cat = jnp.concatenate([Wl1, Wr1], axis=1)
    h1w, h1r0, h1r1 = _expert_mats(xcat, W1cat, bl1[:, None, :], Wl2, Wr2,
                                   oh0, oh1)

    h1w_flat = h1w.reshape(NE * N, D)
    g0 = e0[dst] * N + src
    g1 = e1[dst] * N + src
    a0 = jax.ops.segment_sum(jnp.take(h1w_flat, g0, axis=0), dst,
                             num_segments=N)
    a1 = jax.ops.segment_sum(jnp.take(h1w_flat, g1, axis=0), dst,
                             num_segments=N)

    return _combine(a0, a1, rdeg, p, oh0, oh1, bl2, h1r0, h1r1)


# speedup vs baseline: 1.3096x; 1.0004x over previous
import jax
import jax.numpy as jnp
from jax.experimental import pallas as pl

N = 10000
E = 160000
D = 256
NE = 8
RB = 1000

_INTERP = False


def _gate_body(x_ref, wg_ref, bg_ref, p_ref, oh0_ref, oh1_ref, ep_ref):
    logits = jnp.dot(x_ref[...], wg_ref[...],
                     preferred_element_type=jnp.float32) + bg_ref[...]
    m = jnp.max(logits, axis=1, keepdims=True)
    ex = jnp.exp(logits - m)
    p = ex / jnp.sum(ex, axis=1, keepdims=True)
    iota = jax.lax.broadcasted_iota(jnp.int32, p.shape, 1)
    m0 = jnp.max(p, axis=1, keepdims=True)
    i0 = jnp.min(jnp.where(p == m0, iota, NE + 1), axis=1, keepdims=True)
    oh0 = (iota == i0).astype(jnp.float32)
    p1 = jnp.where(iota == i0, -1.0, p)
    m1 = jnp.max(p1, axis=1, keepdims=True)
    i1 = jnp.min(jnp.where(p1 == m1, iota, NE + 1), axis=1, keepdims=True)
    oh1 = (iota == i1).astype(jnp.float32)
    p_ref[...] = p
    oh0_ref[...] = oh0
    oh1_ref[...] = oh1
    ep_ref[...] = jnp.where(iota == 0, i0, jnp.where(iota == 1, i1, 0))


def _gate(x, Wg, bg):
    nb = N // RB
    return pl.pallas_call(
        _gate_body,
        grid=(nb,),
        in_specs=[
            pl.BlockSpec((RB, D), lambda r: (r, 0)),
            pl.BlockSpec((D, NE), lambda r: (0, 0)),
            pl.BlockSpec((NE,), lambda r: (0,)),
        ],
        out_specs=[
            pl.BlockSpec((RB, NE), lambda r: (r, 0)),
            pl.BlockSpec((RB, NE), lambda r: (r, 0)),
            pl.BlockSpec((RB, NE), lambda r: (r, 0)),
            pl.BlockSpec((RB, NE), lambda r: (r, 0)),
        ],
        out_shape=[
            jax.ShapeDtypeStruct((N, NE), jnp.float32),
            jax.ShapeDtypeStruct((N, NE), jnp.float32),
            jax.ShapeDtypeStruct((N, NE), jnp.float32),
            jax.ShapeDtypeStruct((N, NE), jnp.int32),
        ],
        interpret=_INTERP,
    )(x, Wg, bg)


def _expert_body(xcat_ref, w1_ref, b1_ref, wl2_ref, wr2_ref, oh0_ref, oh1_ref,
                 h1w_ref, h1r0_ref, h1r1_ref):
    e = pl.program_id(1)
    h1 = jnp.dot(xcat_ref[...].astype(jnp.bfloat16),
                 w1_ref[0].astype(jnp.bfloat16),
                 preferred_element_type=jnp.float32) + b1_ref[0]
    h1 = jnp.maximum(h1, 0.0).astype(jnp.bfloat16)
    h1w_ref[0] = jnp.dot(h1, wl2_ref[0].astype(jnp.bfloat16),
                         preferred_element_type=jnp.float32)
    hr = jnp.dot(h1, wr2_ref[0].astype(jnp.bfloat16),
                 preferred_element_type=jnp.float32)
    iota = jax.lax.broadcasted_iota(jnp.int32, oh0_ref.shape, 1)
    sel = (iota == e).astype(jnp.float32)
    m0 = jnp.sum(oh0_ref[...] * sel, axis=1, keepdims=True)
    m1 = jnp.sum(oh1_ref[...] * sel, axis=1, keepdims=True)

    @pl.when(e == 0)
    def _():
        h1r0_ref[...] = m0 * hr
        h1r1_ref[...] = m1 * hr

    @pl.when(e > 0)
    def _():
        h1r0_ref[...] += m0 * hr
        h1r1_ref[...] += m1 * hr


def _expert_mats(xcat, W1cat, bl1, Wl2, Wr2, oh0, oh1):
    nb = N // RB
    return pl.pallas_call(
        _expert_body,
        grid=(nb, NE),
        in_specs=[
            pl.BlockSpec((RB, 2 * D), lambda r, e: (r, 0)),
            pl.BlockSpec((1, 2 * D, D), lambda r, e: (e, 0, 0)),
            pl.BlockSpec((1, 1, D), lambda r, e: (e, 0, 0)),
            pl.BlockSpec((1, D, D), lambda r, e: (e, 0, 0)),
            pl.BlockSpec((1, D, D), lambda r, e: (e, 0, 0)),
            pl.BlockSpec((RB, NE), lambda r, e: (r, 0)),
            pl.BlockSpec((RB, NE), lambda r, e: (r, 0)),
        ],
        out_specs=[
            pl.BlockSpec((1, RB, D), lambda r, e: (e, r, 0)),
            pl.BlockSpec((RB, D), lambda r, e: (r, 0)),
            pl.BlockSpec((RB, D), lambda r, e: (r, 0)),
        ],
        out_shape=[
            jax.ShapeDtypeStruct((NE, N, D), jnp.float32),
            jax.ShapeDtypeStruct((N, D), jnp.float32),
            jax.ShapeDtypeStruct((N, D), jnp.float32),
        ],
        interpret=_INTERP,
    )(xcat, W1cat, bl1, Wl2, Wr2, oh0, oh1)


def _combine_body(a0_ref, a1_ref, rdeg_ref, p_ref, oh0_ref, oh1_ref, bl2_ref,
                  h1r0_ref, h1r1_ref, out_ref):
    rdeg = rdeg_ref[...]
    w0 = jnp.sum(p_ref[...] * oh0_ref[...], axis=1, keepdims=True)
    w1 = jnp.sum(p_ref[...] * oh1_ref[...], axis=1, keepdims=True)
    b0 = jnp.dot(oh0_ref[...], bl2_ref[...], preferred_element_type=jnp.float32)
    b1 = jnp.dot(oh1_ref[...], bl2_ref[...], preferred_element_type=jnp.float32)
    o0 = jnp.maximum(a0_ref[...] * rdeg + b0 + h1r0_ref[...], 0.0)
    o1 = jnp.maximum(a1_ref[...] * rdeg + b1 + h1r1_ref[...], 0.0)
    out_ref[...] = w0 * o0 + w1 * o1


def _combine(a0, a1, rdeg, p, oh0, oh1, bl2, h1r0, h1r1):
    nb = N // RB
    return pl.pallas_call(
        _combine_body,
        grid=(nb,),
        in_specs=[
            pl.BlockSpec((RB, D), lambda r: (r, 0)),
            pl.BlockSpec((RB, D), lambda r: (r, 0)),
            pl.BlockSpec((RB, 1), lambda r: (r, 0)),
            pl.BlockSpec((RB, NE), lambda r: (r, 0)),
            pl.BlockSpec((RB, NE), lambda r: (r, 0)),
            pl.BlockSpec((RB, NE), lambda r: (r, 0)),
            pl.BlockSpec((NE, D), lambda r: (0, 0)),
            pl.BlockSpec((RB, D), lambda r: (r, 0)),
            pl.BlockSpec((RB, D), lambda r: (r, 0)),
        ],
        out_specs=pl.BlockSpec((RB, D), lambda r: (r, 0)),
        out_shape=jax.ShapeDtypeStruct((N, D), jnp.float32),
        interpret=_INTERP,
    )(a0, a1, rdeg, p, oh0, oh1, bl2, h1r0, h1r1)


def kernel(x, edge_index, Wg, bg, Wl1, bl1, Wr1, Wl2, bl2, Wr2):
    src = edge_index[0]
    dst = edge_index[1]

    p, oh0, oh1, ep = _gate(x, Wg, bg)
    e0 = ep[:, 0]
    e1 = ep[:, 1]

    ones = jnp.ones((E,), jnp.float32)
    deg = jax.ops.segment_sum(ones, dst, num_segments=N)
    aggx = jax.ops.segment_sum(jnp.take(x, src, axis=0), dst, num_segments=N)
    rdeg = (1.0 / jnp.maximum(deg, 1.0))[:, None]
    meanx = aggx * rdeg

    xcat = jnp.concatenate([meanx, x], axis=1)
    W1cat = jnp.concatenate([Wl1, Wr1], axis=1)
    h1w, h1r0, h1r1 = _expert_mats(xcat, W1cat, bl1[:, None, :], Wl2, Wr2,
                                   oh0, oh1)

    h1w_flat = h1w.reshape(NE * N, D)
    g0 = e0[dst] * N + src
    g1 = e1[dst] * N + src
    a0 = jax.ops.segment_sum(jnp.take(h1w_flat, g0, axis=0), dst,
                             num_segments=N)
    a1 = jax.ops.segment_sum(jnp.take(h1w_flat, g1, axis=0), dst,
                             num_segments=N)

    return _combine(a0, a1, rdeg, p, oh0, oh1, bl2, h1r0, h1r1)


# speedup vs baseline: 2.0623x; 1.5747x over previous
import jax
import jax.numpy as jnp
from jax.experimental import pallas as pl

N = 10000
E = 160000
D = 256
NE = 8
RB = 1000

_INTERP = False


def _gate_body(x_ref, wg_ref, bg_ref, p_ref, oh0_ref, oh1_ref, ep_ref):
    logits = jnp.dot(x_ref[...], wg_ref[...],
                     preferred_element_type=jnp.float32) + bg_ref[...]
    m = jnp.max(logits, axis=1, keepdims=True)
    ex = jnp.exp(logits - m)
    p = ex / jnp.sum(ex, axis=1, keepdims=True)
    iota = jax.lax.broadcasted_iota(jnp.int32, p.shape, 1)
    m0 = jnp.max(p, axis=1, keepdims=True)
    i0 = jnp.min(jnp.where(p == m0, iota, NE + 1), axis=1, keepdims=True)
    oh0 = (iota == i0).astype(jnp.float32)
    p1 = jnp.where(iota == i0, -1.0, p)
    m1 = jnp.max(p1, axis=1, keepdims=True)
    i1 = jnp.min(jnp.where(p1 == m1, iota, NE + 1), axis=1, keepdims=True)
    oh1 = (iota == i1).astype(jnp.float32)
    p_ref[...] = p
    oh0_ref[...] = oh0
    oh1_ref[...] = oh1
    ep_ref[...] = jnp.where(iota == 0, i0, jnp.where(iota == 1, i1, 0))


def _gate(x, Wg, bg):
    nb = N // RB
    return pl.pallas_call(
        _gate_body,
        grid=(nb,),
        in_specs=[
            pl.BlockSpec((RB, D), lambda r: (r, 0)),
            pl.BlockSpec((D, NE), lambda r: (0, 0)),
            pl.BlockSpec((NE,), lambda r: (0,)),
        ],
        out_specs=[
            pl.BlockSpec((RB, NE), lambda r: (r, 0)),
            pl.BlockSpec((RB, NE), lambda r: (r, 0)),
            pl.BlockSpec((RB, NE), lambda r: (r, 0)),
            pl.BlockSpec((RB, NE), lambda r: (r, 0)),
        ],
        out_shape=[
            jax.ShapeDtypeStruct((N, NE), jnp.float32),
            jax.ShapeDtypeStruct((N, NE), jnp.float32),
            jax.ShapeDtypeStruct((N, NE), jnp.float32),
            jax.ShapeDtypeStruct((N, NE), jnp.int32),
        ],
        interpret=_INTERP,
    )(x, Wg, bg)


def _expert_body(xcat_ref, w1_ref, b1_ref, wl2_ref, wr2_ref, oh0_ref, oh1_ref,
                 h1w_ref, h1r0_ref, h1r1_ref):
    e = pl.program_id(1)
    h1 = jnp.dot(xcat_ref[...], w1_ref[0],
                 preferred_element_type=jnp.float32) + b1_ref[0]
    h1 = jnp.maximum(h1, 0.0)
    h1w_ref[0] = jnp.dot(h1, wl2_ref[0], preferred_element_type=jnp.float32)
    hr = jnp.dot(h1, wr2_ref[0], preferred_element_type=jnp.float32)
    iota = jax.lax.broadcasted_iota(jnp.int32, oh0_ref.shape, 1)
    sel = (iota == e).astype(jnp.float32)
    m0 = jnp.sum(oh0_ref[...] * sel, axis=1, keepdims=True)
    m1 = jnp.sum(oh1_ref[...] * sel, axis=1, keepdims=True)

    @pl.when(e == 0)
    def _():
        h1r0_ref[...] = m0 * hr
        h1r1_ref[...] = m1 * hr

    @pl.when(e > 0)
    def _():
        h1r0_ref[...] += m0 * hr
        h1r1_ref[...] += m1 * hr


def _expert_mats(xcat, W1cat, bl1, Wl2, Wr2, oh0, oh1):
    nb = N // RB
    return pl.pallas_call(
        _expert_body,
        grid=(nb, NE),
        in_specs=[
            pl.BlockSpec((RB, 2 * D), lambda r, e: (r, 0)),
            pl.BlockSpec((1, 2 * D, D), lambda r, e: (e, 0, 0)),
            pl.BlockSpec((1, 1, D), lambda r, e: (e, 0, 0)),
            pl.BlockSpec((1, D, D), lambda r, e: (e, 0, 0)),
            pl.BlockSpec((1, D, D), lambda r, e: (e, 0, 0)),
            pl.BlockSpec((RB, NE), lambda r, e: (r, 0)),
            pl.BlockSpec((RB, NE), lambda r, e: (r, 0)),
        ],
        out_specs=[
            pl.BlockSpec((1, RB, D), lambda r, e: (e, r, 0)),
            pl.BlockSpec((RB, D), lambda r, e: (r, 0)),
            pl.BlockSpec((RB, D), lambda r, e: (r, 0)),
        ],
        out_shape=[
            jax.ShapeDtypeStruct((NE, N, D), jnp.float32),
            jax.ShapeDtypeStruct((N, D), jnp.float32),
            jax.ShapeDtypeStruct((N, D), jnp.float32),
        ],
        interpret=_INTERP,
    )(xcat, W1cat, bl1, Wl2, Wr2, oh0, oh1)


def _combine_body(a0_ref, a1_ref, rdeg_ref, p_ref, oh0_ref, oh1_ref, bl2_ref,
                  h1r0_ref, h1r1_ref, out_ref):
    rdeg = rdeg_ref[...]
    w0 = jnp.sum(p_ref[...] * oh0_ref[...], axis=1, keepdims=True)
    w1 = jnp.sum(p_ref[...] * oh1_ref[...], axis=1, keepdims=True)
    b0 = jnp.dot(oh0_ref[...], bl2_ref[...], preferred_element_type=jnp.float32)
    b1 = jnp.dot(oh1_ref[...], bl2_ref[...], preferred_element_type=jnp.float32)
    o0 = jnp.maximum(a0_ref[...] * rdeg + b0 + h1r0_ref[...], 0.0)
    o1 = jnp.maximum(a1_ref[...] * rdeg + b1 + h1r1_ref[...], 0.0)
    out_ref[...] = w0 * o0 + w1 * o1


def _combine(a0, a1, rdeg, p, oh0, oh1, bl2, h1r0, h1r1):
    nb = N // RB
    return pl.pallas_call(
        _combine_body,
        grid=(nb,),
        in_specs=[
            pl.BlockSpec((RB, D), lambda r: (r, 0)),
            pl.BlockSpec((RB, D), lambda r: (r, 0)),
            pl.BlockSpec((RB, 1), lambda r: (r, 0)),
            pl.BlockSpec((RB, NE), lambda r: (r, 0)),
            pl.BlockSpec((RB, NE), lambda r: (r, 0)),
            pl.BlockSpec((RB, NE), lambda r: (r, 0)),
            pl.BlockSpec((NE, D), lambda r: (0, 0)),
            pl.BlockSpec((RB, D), lambda r: (r, 0)),
            pl.BlockSpec((RB, D), lambda r: (r, 0)),
        ],
        out_specs=pl.BlockSpec((RB, D), lambda r: (r, 0)),
        out_shape=jax.ShapeDtypeStruct((N, D), jnp.float32),
        interpret=_INTERP,
    )(a0, a1, rdeg, p, oh0, oh1, bl2, h1r0, h1r1)


def kernel(x, edge_index, Wg, bg, Wl1, bl1, Wr1, Wl2, bl2, Wr2):
    src = edge_index[0]
    dst = edge_index[1]

    p, oh0, oh1, ep = _gate(x, Wg, bg)

    ones = jnp.ones((E,), jnp.float32)
    deg = jax.ops.segment_sum(ones, dst, num_segments=N)
    aggx = jax.ops.segment_sum(jnp.take(x, src, axis=0), dst, num_segments=N)
    rdeg = (1.0 / jnp.maximum(deg, 1.0))[:, None]
    meanx = aggx * rdeg

    xcat = jnp.concatenate([meanx, x], axis=1)
    W1cat = jnp.concatenate([Wl1, Wr1], axis=1)
    h1w, h1r0, h1r1 = _expert_mats(xcat, W1cat, bl1[:, None, :], Wl2, Wr2,
                                   oh0, oh1)

    h1w_flat = h1w.reshape(NE * N, D)
    eb = jnp.take(ep[:, :2] * N, dst, axis=0)
    gg = jnp.concatenate([eb[:, 0] + src, eb[:, 1] + src])
    segid = jnp.concatenate([dst, dst + N])
    y = jax.ops.segment_sum(jnp.take(h1w_flat, gg, axis=0), segid,
                            num_segments=2 * N)
    a0 = y[:N]
    a1 = y[N:]

    return _combine(a0, a1, rdeg, p, oh0, oh1, bl2, h1r0, h1r1)
